# Initial kernel scaffold; baseline (speedup 1.0000x reference)
import jax
import jax.numpy as jnp
from jax.experimental import pallas as pl


def kernel(cart, input_embed, contracted_coeff, ens_cg, p_neigh, p_cg, p_mp, p_out, neighlist, index_l, index_i1, index_i2, index_add, index_cg):
    raise NotImplementedError("write your pallas kernel here")


# speedup vs baseline: 23.1781x; 23.1781x over previous
import math

import jax
import jax.numpy as jnp
from jax import lax
from jax.experimental import pallas as pl

N = 192
M = N - 1
P = N * M
NW = 8
NC = 32
CB = 8
PB = CB * M
NBLK = N // CB
C0 = 0.28209479177387814
C1 = 0.4886025119029199
F32 = jnp.float32
_INTERP = False


def _silu(x):
    return x * jax.nn.sigmoid(x)


def _exp128(a):
    hi = a[:, 32:64]
    return jnp.concatenate([a[:, 0:32], hi, hi, hi], axis=1)


def _seg_matrix():
    ia = lax.broadcasted_iota(jnp.int32, (CB, PB), 0)
    ir = lax.broadcasted_iota(jnp.int32, (CB, PB), 1)
    lo = ia * M
    return jnp.where((ir >= lo) & (ir < lo + M), F32(1.0), F32(0.0))


def _onehot_pairs(idx_col):
    it = lax.broadcasted_iota(jnp.int32, (PB, N), 1)
    return jnp.where(idx_col == it, F32(1.0), F32(0.0))


def _pair_kernel(ids, emb, cart, cc0, cc1, cc2,
                 nw1, nb1, nw2, nb2, nw3, nb3,
                 cw1, cb1, cw2, cb2, cw3, cb3,
                 co_ref, rs1_ref, rs2_ref, cg_ref):
    dot = lambda a, b: jnp.dot(a, b, preferred_element_type=F32)
    x = emb[...]
    h = _silu(dot(x, nw1[...]) + nb1[...])
    h = _silu(dot(h, nw2[...]) + nb2[...])
    e = dot(h, nw3[...]) + nb3[...]

    idsv = ids[...]
    ohc = _onehot_pairs(idsv[:, 0:1])
    ohn = _onehot_pairs(idsv[:, 1:2])
    cartv = cart[...]
    dv = dot(ohn, cartv) - dot(ohc, cartv)
    d2 = jnp.sum(dv * dv, axis=1, keepdims=True)
    dist = jnp.sqrt(d2)

    w = e[:, 0:NW]
    cen = e[:, NW:2 * NW]
    t = w * (dist - cen)
    rad8 = jnp.exp(-(t * t))
    r0 = dot(rad8, cc0[...])
    r1 = dot(rad8, cc1[...])
    r2 = dot(rad8, cc2[...])

    u = dv / dist
    ones32 = jnp.ones((PB, NC), dtype=F32)
    sph = jnp.concatenate([C0 * ones32,
                           (C1 * u[:, 1:2]) * ones32,
                           (C1 * u[:, 2:3]) * ones32,
                           (C1 * u[:, 0:1]) * ones32], axis=1)

    wr0 = e[:, 2 * NW:] * r0
    worb = _exp128(wr0) * sph
    rs1_ref[...] = _exp128(r1) * sph
    rs2_ref[...] = _exp128(r2) * sph

    h2 = _silu(dot(x, cw1[...]) + cb1[...])
    h2 = _silu(dot(h2, cw2[...]) + cb2[...])
    cg_ref[...] = dot(h2, cw3[...]) + cb3[...]

    co_ref[...] = dot(_seg_matrix(), worb) * F32(1.0 / M)


def _inter_kernel(ids, rs, cg, co_tab, cf_tab, q1, q2, rsel, aadd, co_ref):
    dot = lambda a, b: jnp.dot(a, b, preferred_element_type=F32)
    ohn = _onehot_pairs(ids[...][:, 1:2])
    g = dot(ohn, co_tab[...])
    cf = dot(ohn, cf_tab[...])
    orbital = rs[...] * _exp128(cf)
    io1 = dot(g, q1[...])
    io2 = dot(orbital, q2[...])
    cge = dot(cg[...], rsel[...])
    inter = io1 * io2 * cge
    wo = dot(inter, aadd[...])
    co_ref[...] = dot(_seg_matrix(), wo) * F32(1.0 / M)


def _density_add(co):
    sq = co * co
    return jnp.concatenate(
        [sq[:, 0:NC], sq[:, NC:2 * NC] + sq[:, 2 * NC:3 * NC] + sq[:, 3 * NC:]],
        axis=1)


def _atom_kernel(co, dprev, w1, b1, w2, b2, w3, b3, dens_ref, coeff_ref, *, scale):
    dot = lambda a, b: jnp.dot(a, b, preferred_element_type=F32)
    dens = (dprev[...] + _density_add(co[...])) * F32(scale)
    dens_ref[...] = dens
    h = _silu(dot(dens, w1[...]) + b1[...])
    h = _silu(dot(h, w2[...]) + b2[...])
    coeff_ref[...] = dot(h, w3[...]) + b3[...]


def _final_kernel(co, dprev, w1, b1, w2, b2, w3, b3, out_ref):
    dot = lambda a, b: jnp.dot(a, b, preferred_element_type=F32)
    dens = (dprev[...] + _density_add(co[...])) * F32(1.0 / math.sqrt(2.0))
    h = _silu(dot(dens, w1[...]) + b1[...])
    h = _silu(dot(h, w2[...]) + b2[...])
    y = dot(h, w3[...]) + b3[...]
    out_ref[...] = jnp.sum(y, axis=0, keepdims=True)


def _full(shape):
    nd = len(shape)
    return pl.BlockSpec(shape, lambda i: (0,) * nd)


def kernel(cart, input_embed, contracted_coeff, ens_cg, p_neigh, p_cg, p_mp, p_out,
           neighlist, index_l, index_i1, index_i2, index_add, index_cg):
    f32 = lambda a: a.astype(F32)
    ids = neighlist.T.astype(jnp.int32)
    cc = f32(contracted_coeff).reshape(3, 2 * NC, NW).transpose(0, 2, 1)

    def mlp_args(p):
        w1, b1, w2, b2, w3, b3 = p
        return [f32(w1), f32(b1).reshape(1, -1), f32(w2), f32(b2).reshape(1, -1),
                f32(w3), f32(b3).reshape(1, -1)]

    eye32 = jnp.eye(NC, dtype=F32)
    oh1 = jax.nn.one_hot(index_i1, 4, dtype=F32)
    oh2 = jax.nn.one_hot(index_i2, 4, dtype=F32)
    ohadd = jax.nn.one_hot(index_add, 4, dtype=F32)
    q1 = jnp.kron(oh1.T, eye32)
    q2 = jnp.kron(oh2.T, eye32)
    aadd = jnp.kron(ohadd * f32(ens_cg)[:, None], eye32)
    eye24 = jnp.eye(24, dtype=F32)
    rsel = [jnp.kron(eye24[:, 12 * it:12 * (it + 1)], jnp.ones((1, NC), dtype=F32))
            for it in range(2)]

    pair_specs = (
        [pl.BlockSpec((PB, 2), lambda i: (i, 0)),
         pl.BlockSpec((PB, 16), lambda i: (i, 0)),
         _full((N, 3)), _full((NW, 64)), _full((NW, 64)), _full((NW, 64))]
        + [_full(s.shape) for s in mlp_args(p_neigh)]
        + [_full(s.shape) for s in mlp_args(p_cg)]
    )
    co0, rs1, rs2, cg24 = pl.pallas_call(
        _pair_kernel,
        grid=(NBLK,),
        in_specs=pair_specs,
        out_specs=[pl.BlockSpec((CB, 128), lambda i: (i, 0)),
                   pl.BlockSpec((PB, 128), lambda i: (i, 0)),
                   pl.BlockSpec((PB, 128), lambda i: (i, 0)),
                   pl.BlockSpec((PB, 24), lambda i: (i, 0))],
        out_shape=[jax.ShapeDtypeStruct((N, 128), F32),
                   jax.ShapeDtypeStruct((P, 128), F32),
                   jax.ShapeDtypeStruct((P, 128), F32),
                   jax.ShapeDtypeStruct((P, 24), F32)],
        interpret=_INTERP,
    )(ids, f32(input_embed), f32(cart), cc[0], cc[1], cc[2],
      *mlp_args(p_neigh), *mlp_args(p_cg))

    def atom_step(co, dprev, params, scale):
        fn = lambda *refs: _atom_kernel(*refs, scale=scale)
        return pl.pallas_call(
            fn,
            grid=(1,),
            in_specs=[_full((N, 128)), _full((N, 64))] + [_full(s.shape) for s in mlp_args(params)],
            out_specs=[_full((N, 64)), _full((N, 64))],
            out_shape=[jax.ShapeDtypeStruct((N, 64), F32),
                       jax.ShapeDtypeStruct((N, 64), F32)],
            interpret=_INTERP,
        )(co, dprev, *mlp_args(params))

    def inter_step(rs, it, co_tab, cf_tab):
        return pl.pallas_call(
            _inter_kernel,
            grid=(NBLK,),
            in_specs=[pl.BlockSpec((PB, 2), lambda i: (i, 0)),
                      pl.BlockSpec((PB, 128), lambda i: (i, 0)),
                      pl.BlockSpec((PB, 24), lambda i: (i, 0)),
                      _full((N, 128)), _full((N, 64)),
                      _full((128, 384)), _full((128, 384)),
                      _full((24, 384)), _full((384, 128))],
            out_specs=pl.BlockSpec((CB, 128), lambda i: (i, 0)),
            out_shape=jax.ShapeDtypeStruct((N, 128), F32),
            interpret=_INTERP,
        )(ids, rs, cg24, co_tab, cf_tab, q1, q2, rsel[it], aadd)

    zeros64 = jnp.zeros((N, 64), dtype=F32)
    dens0, coeff0 = atom_step(co0, zeros64, p_mp[0], 1.0)
    co1 = inter_step(rs1, 0, co0, coeff0)
    dens1, coeff1 = atom_step(co1, dens0, p_mp[1], 1.0 / math.sqrt(2.0))
    co2 = inter_step(rs2, 1, co1, coeff1)

    out = pl.pallas_call(
        _final_kernel,
        grid=(1,),
        in_specs=[_full((N, 128)), _full((N, 64))] + [_full(s.shape) for s in mlp_args(p_out)],
        out_specs=_full((1, 1)),
        out_shape=jax.ShapeDtypeStruct((1, 1), F32),
        interpret=_INTERP,
    )(co2, dens1, *mlp_args(p_out))
    return out[0, 0]
